# baseline (speedup 1.0000x reference)
import functools

import jax
import jax.numpy as jnp
from jax import lax
from jax.experimental import pallas as pl
from jax.experimental.pallas import tpu as pltpu
from jax.experimental.pallas import tpu_sc as plsc

_D = 64
_B = 4096 * 200
_NW = 32
_CHUNK = 128
_CPW = _B // _NW // _CHUNK
_NBUF = 4
_NGROUPS = _CPW // _NBUF


def _make_embed():
    mesh = plsc.VectorSubcoreMesh(core_axis_name="c", subcore_axis_name="s")

    @functools.partial(
        pl.kernel,
        mesh=mesh,
        out_type=jax.ShapeDtypeStruct((_B, _D), jnp.float32),
        scratch_types=[
            pltpu.VMEM((_CPW, _CHUNK), jnp.int32),
            pltpu.VMEM((_NBUF, _CHUNK, _D), jnp.float32),
        ] + [pltpu.SemaphoreType.DMA] * (2 * _NBUF),
        compiler_params=pltpu.CompilerParams(use_tc_tiling_on_sc=False),
    )
    def embed(x_hbm, table_hbm, out_hbm, idx_v, rows_v, *sems):
        sem_g = sems[:_NBUF]
        sem_o = sems[_NBUF:]
        wid = lax.axis_index("s") * 2 + lax.axis_index("c")
        row0 = wid * _CPW
        out0 = wid * _CPW * _CHUNK

        pltpu.sync_copy(x_hbm.at[pl.ds(row0, _CPW)], idx_v)

        def g_copy(ch, b):
            return pltpu.make_async_copy(
                table_hbm.at[idx_v.at[ch]], rows_v.at[b], sem_g[b])

        def o_copy(ch, b):
            return pltpu.make_async_copy(
                rows_v.at[b], out_hbm.at[pl.ds(out0 + ch * _CHUNK, _CHUNK)],
                sem_o[b])

        for b in range(_NBUF):
            g_copy(b, b).start()

        def group(gi, carry):
            g = gi * _NBUF
            for b in range(_NBUF):
                g_copy(g + b, b).wait()
                o_copy(g + b, b).start()
            for b in range(_NBUF):
                o_copy(g + b, b).wait()
                g_copy(g + _NBUF + b, b).start()
            return carry

        lax.fori_loop(0, _NGROUPS - 1, group, 0)

        gl = (_NGROUPS - 1) * _NBUF
        for b in range(_NBUF):
            g_copy(gl + b, b).wait()
            o_copy(gl + b, b).start()
        for b in range(_NBUF):
            o_copy(gl + b, b).wait()

    return embed


_embed = _make_embed()


def kernel(x, table):
    x2d = x.reshape(_B // _CHUNK, _CHUNK)
    out = _embed(x2d, table)
    return out.reshape(x.shape[0], x.shape[1], _D)
